# baseline (speedup 1.0000x reference)
import functools

import jax
import jax.numpy as jnp
from jax import lax
from jax.experimental import pallas as pl
from jax.experimental.pallas import tpu as pltpu
from jax.experimental.pallas import tpu_sc as plsc

NUM_FEATURES = 26
CAT_SIZE = 1000
D_EMBED = 128
BATCH = 16384
PAD_F = 32

NC = 2
NS = 16
NW = NC * NS
SAMP_W = BATCH // NW
SAMP_C = 16
N_CHUNKS = SAMP_W // SAMP_C


def _sc_embedding_gather(x_cat, weight):
    mesh = plsc.VectorSubcoreMesh(core_axis_name="c", subcore_axis_name="s")

    @functools.partial(
        pl.kernel,
        mesh=mesh,
        out_type=jax.ShapeDtypeStruct((BATCH, NUM_FEATURES, D_EMBED), jnp.float32),
        scratch_types=[
            pltpu.VMEM((SAMP_C, NUM_FEATURES), jnp.int32),
            pltpu.VMEM((SAMP_C, PAD_F), jnp.int32),
            pltpu.VMEM((SAMP_C, PAD_F, D_EMBED), jnp.float32),
            pltpu.SemaphoreType.DMA,
        ],
    )
    def body(x_hbm, w_hbm, out_hbm, idx_in, idx32, buf, sem):
        wid = lax.axis_index("s") * NC + lax.axis_index("c")
        base = wid * SAMP_W
        off1 = lax.iota(jnp.int32, 16) * CAT_SIZE
        off2 = (lax.iota(jnp.int32, 16) + 10) * CAT_SIZE
        zeros = jnp.zeros((16,), jnp.int32)

        def chunk_body(c, carry):
            sb = pl.multiple_of(base + c * SAMP_C, SAMP_C)
            pltpu.sync_copy(x_hbm.at[pl.ds(sb, SAMP_C)], idx_in)
            for i in range(SAMP_C):
                v1 = idx_in[i, pl.ds(0, 16)]
                v2 = idx_in[i, pl.ds(10, 16)]
                idx32[i, pl.ds(16, 16)] = zeros
                idx32[i, pl.ds(0, 16)] = v1 + off1
                idx32[i, pl.ds(10, 16)] = v2 + off2
            copies = [
                pltpu.async_copy(w_hbm.at[idx32.at[i]], buf.at[i], sem)
                for i in range(SAMP_C)
            ]
            for cp in copies:
                cp.wait()
            pltpu.sync_copy(
                buf.at[:, pl.ds(0, NUM_FEATURES)],
                out_hbm.at[pl.ds(sb, SAMP_C)],
            )
            return carry

        lax.fori_loop(0, N_CHUNKS, chunk_body, 0)

    return body(x_cat, weight)


def kernel(x_cat, weight):
    return _sc_embedding_gather(x_cat, weight)
